# Initial kernel scaffold; baseline (speedup 1.0000x reference)
import jax
import jax.numpy as jnp
from jax.experimental import pallas as pl


def kernel(x, edge_index, edge_weight, W1, b1, W2, b2):
    raise NotImplementedError("write your pallas kernel here")


# speedup vs baseline: 4.5726x; 4.5726x over previous
import functools

import jax
import jax.numpy as jnp
from jax import lax
from jax.experimental import pallas as pl
from jax.experimental.pallas import tpu as pltpu
from jax.experimental.pallas import tpu_sc as plsc

N = 10000
E = 320000
F_IN = 128
H = 128
C = 40
CP = 48

NC, NS, L = 2, 16, 16
NW = NC * NS
EPT = E // NW
K = 80
NCHUNK = EPT // K
NP = 10240
RPT = NP // NS


@functools.lru_cache(maxsize=None)
def _make_spmm(F):
  mesh = plsc.VectorSubcoreMesh(
      core_axis_name="c", subcore_axis_name="s",
      num_cores=NC, num_subcores=NS)

  @functools.partial(
      pl.kernel,
      out_type=jax.ShapeDtypeStruct((NC, NP, F), jnp.float32),
      mesh=mesh,
      scratch_types=[
          pltpu.VMEM((K,), jnp.int32),
          pltpu.VMEM((K,), jnp.int32),
          pltpu.VMEM((K,), jnp.float32),
          pltpu.VMEM((K, F), jnp.float32),
          pltpu.VMEM_SHARED((NP, F), jnp.float32),
          pltpu.SemaphoreType.DMA,
      ],
      compiler_params=pltpu.CompilerParams(use_tc_tiling_on_sc=False),
  )
  def spmm(row_hbm, col_hbm, w_hbm, sup_hbm, zero_hbm, out_hbm,
           colbuf, rowbuf, wbuf, rows, acc, sem):
    c = lax.axis_index("c")
    s = lax.axis_index("s")
    wid = s * NC + c
    base = wid * EPT

    pltpu.sync_copy(zero_hbm.at[pl.ds(s * RPT, RPT)],
                    acc.at[pl.ds(s * RPT, RPT)])
    plsc.subcore_barrier()

    @pl.loop(0, NCHUNK)
    def _chunk(g):
      off = base + g * K
      pltpu.sync_copy(col_hbm.at[pl.ds(off, K)], colbuf)
      pltpu.sync_copy(w_hbm.at[pl.ds(off, K)], wbuf)
      pltpu.async_copy(sup_hbm.at[colbuf], rows, sem).wait()

      @pl.loop(0, K // L)
      def _scale(t):
        wv = wbuf[pl.ds(t * L, L)]
        for i in range(L):
          e = t * L + i
          w = wv[i]
          for j in range(F // L):
            sl = pl.ds(j * L, L)
            rows[e, sl] = rows[e, sl] * w

      pltpu.sync_copy(row_hbm.at[pl.ds(off, K)], rowbuf)
      pltpu.sync_copy(rows, acc.at[rowbuf], add=True)

    plsc.subcore_barrier()
    pltpu.sync_copy(acc.at[pl.ds(s * RPT, RPT)],
                    out_hbm.at[c, pl.ds(s * RPT, RPT)])

  return spmm


def _tc1_body(x_ref, w_ref, o_ref):
  o_ref[...] = jnp.dot(x_ref[...], w_ref[...],
                       preferred_element_type=jnp.float32)


def _tc2_body(p_ref, b1_ref, w2_ref, o_ref):
  h = jnp.maximum(p_ref[0, :N] + p_ref[1, :N] + b1_ref[...], 0.0)
  o_ref[...] = jnp.dot(h, w2_ref[...], preferred_element_type=jnp.float32)


def _tc3_body(q_ref, b2_ref, o_ref):
  logits = q_ref[0, :N, :C] + q_ref[1, :N, :C] + b2_ref[...]
  m = jnp.max(logits, axis=1, keepdims=True)
  ex = jnp.exp(logits - m)
  lse = jnp.log(jnp.sum(ex, axis=1, keepdims=True))
  o_ref[...] = logits - m - lse


_tc1 = pl.pallas_call(
    _tc1_body, out_shape=jax.ShapeDtypeStruct((N, H), jnp.float32))
_tc2 = pl.pallas_call(
    _tc2_body, out_shape=jax.ShapeDtypeStruct((N, CP), jnp.float32))
_tc3 = pl.pallas_call(
    _tc3_body, out_shape=jax.ShapeDtypeStruct((N, C), jnp.float32))


def kernel(x, edge_index, edge_weight, W1, b1, W2, b2):
  row = edge_index[0]
  col = edge_index[1]
  w2p = jnp.pad(W2, ((0, 0), (0, CP - C)))

  support = _tc1(x, W1)
  part1 = _make_spmm(H)(row, col, edge_weight, support,
                        jnp.zeros((NP, H), jnp.float32))
  support2 = _tc2(part1, b1, w2p)
  part2 = _make_spmm(CP)(row, col, edge_weight, support2,
                         jnp.zeros((NP, CP), jnp.float32))
  return _tc3(part2, b2)
